# Initial kernel scaffold; baseline (speedup 1.0000x reference)
import functools

import jax
import jax.numpy as jnp
from jax import lax
from jax.experimental import pallas as pl
from jax.experimental.pallas import tpu as pltpu
from jax.experimental.pallas import tpu_sc as plsc

N = 10000
E = 160000
D = 256
DH = D // 2
NPAD = 10240
ROWS_PER_TILE = 640
SUB = 80
CHUNK = 128
NCHUNKS = E // CHUNK
NBLK = 10
BLK = N // NBLK


def _prep_compute(x, wl_ref, wr_ref, b_ref, yl_ref, yr_ref, zl_ref, zr_ref):
    y = jnp.dot(x, wl_ref[...], preferred_element_type=jnp.float32)
    z = jnp.dot(x, wr_ref[...], preferred_element_type=jnp.float32) + b_ref[...]
    yl_ref[...] = y[:, :DH]
    yr_ref[...] = y[:, DH:]
    zl_ref[...] = z[:, :DH]
    zr_ref[...] = z[:, DH:]


def _prep_body_full(x_ref, wl_ref, wr_ref, b_ref, *out_refs):
    _prep_compute(x_ref[...], wl_ref, wr_ref, b_ref, *out_refs)


def _prep_body_split(x0_ref, x1_ref, wl_ref, wr_ref, b_ref, *out_refs):
    x = jnp.concatenate([x0_ref[...], x1_ref[...]], axis=1)
    _prep_compute(x, wl_ref, wr_ref, b_ref, *out_refs)


def _make_prep(split):
    if split:
        body = _prep_body_split
        x_specs = [pl.BlockSpec((BLK, DH), lambda i: (i, 0)),
                   pl.BlockSpec((BLK, DH), lambda i: (i, 0))]
    else:
        body = _prep_body_full
        x_specs = [pl.BlockSpec((BLK, D), lambda i: (i, 0))]
    half = pl.BlockSpec((BLK, DH), lambda i: (i, 0))
    return pl.pallas_call(
        body,
        grid=(NBLK,),
        in_specs=x_specs + [
            pl.BlockSpec((D, D), lambda i: (0, 0)),
            pl.BlockSpec((D, D), lambda i: (0, 0)),
            pl.BlockSpec((1, D), lambda i: (0, 0)),
        ],
        out_specs=[half, half, half, half],
        out_shape=[jax.ShapeDtypeStruct((N, DH), jnp.float32)] * 4,
    )


def _spmm_body(relu, yl, yr, zl, zr, src_hbm, dst_hbm, zrow, zcnt, ones_hbm,
               out_hbm, src_v, dst_v, rows_v, ones_v, abuf, zbuf, outb, cbuf,
               acc, cnt, sem):
    c = lax.axis_index("c")
    s = lax.axis_index("s")
    row0 = s * ROWS_PER_TILE
    nrows = jnp.minimum(ROWS_PER_TILE, N - row0)
    nsub = nrows // SUB

    def zero_j(j, carry):
        pltpu.sync_copy(zrow, acc.at[pl.ds(row0 + j * SUB, SUB)])
        return carry
    lax.fori_loop(0, nsub, zero_j, 0)

    @pl.when(s == 0)
    def _():
        pltpu.sync_copy(zcnt, cnt)

    pltpu.sync_copy(ones_hbm, ones_v)
    plsc.subcore_barrier()

    nk = (NCHUNKS - s + 15) // 16

    def edge_k(k, carry):
        base = (k * 16 + s) * CHUNK
        pltpu.sync_copy(src_hbm.at[pl.ds(base, CHUNK)], src_v)
        pltpu.sync_copy(dst_hbm.at[pl.ds(base, CHUNK)], dst_v)

        @pl.when(c == 0)
        def _():
            pltpu.async_copy(yl.at[src_v], rows_v, sem).wait()

        @pl.when(c == 1)
        def _():
            pltpu.async_copy(yr.at[src_v], rows_v, sem).wait()

        pltpu.sync_copy(rows_v, acc.at[dst_v], add=True)
        pltpu.sync_copy(ones_v, cnt.at[dst_v], add=True)
        return carry
    lax.fori_loop(0, nk, edge_k, 0)
    plsc.subcore_barrier()

    def out_j(j, carry):
        base = row0 + j * SUB
        pltpu.sync_copy(acc.at[pl.ds(base, SUB)], abuf)
        pltpu.sync_copy(cnt.at[pl.ds(base, SUB)], cbuf)

        @pl.when(c == 0)
        def _():
            pltpu.sync_copy(zl.at[pl.ds(base, SUB)], zbuf)

        @pl.when(c == 1)
        def _():
            pltpu.sync_copy(zr.at[pl.ds(base, SUB)], zbuf)

        def row_r(r, carry2):
            inv = 1.0 / jnp.maximum(cbuf[r], 1.0)

            def vec_v(v, carry3):
                val = abuf[r, pl.ds(v * 16, 16)] * inv + zbuf[r, pl.ds(v * 16, 16)]
                if relu:
                    val = jnp.maximum(val, 0.0)
                outb[r, pl.ds(v * 16, 16)] = val
                return carry3
            lax.fori_loop(0, DH // 16, vec_v, 0)
            return carry2
        lax.fori_loop(0, SUB, row_r, 0)
        pltpu.sync_copy(outb, out_hbm.at[pl.ds(base, SUB), pl.ds(c * DH, DH)])
        return carry
    lax.fori_loop(0, nsub, out_j, 0)


def _make_spmm(relu):
    mesh = plsc.VectorSubcoreMesh(core_axis_name="c", subcore_axis_name="s")
    return pl.kernel(
        functools.partial(_spmm_body, relu),
        out_type=jax.ShapeDtypeStruct((N, D), jnp.float32),
        mesh=mesh,
        scratch_types=[
            pltpu.VMEM((CHUNK,), jnp.int32),
            pltpu.VMEM((CHUNK,), jnp.int32),
            pltpu.VMEM((CHUNK, DH), jnp.float32),
            pltpu.VMEM((CHUNK,), jnp.float32),
            pltpu.VMEM((SUB, DH), jnp.float32),
            pltpu.VMEM((SUB, DH), jnp.float32),
            pltpu.VMEM((SUB, DH), jnp.float32),
            pltpu.VMEM((SUB,), jnp.float32),
            pltpu.VMEM_SHARED((NPAD, DH), jnp.float32),
            pltpu.VMEM_SHARED((NPAD,), jnp.float32),
            pltpu.SemaphoreType.DMA,
        ],
    )


def kernel(x, edge_index, W_l0, b_l0, W_r0, W_l1, b_l1, W_r1):
    src = edge_index[0]
    dst = edge_index[1]
    zrow = jnp.zeros((SUB, DH), jnp.float32)
    zcnt = jnp.zeros((NPAD,), jnp.float32)
    ones = jnp.ones((CHUNK,), jnp.float32)

    yl, yr, zl, zr = _make_prep(split=False)(x, W_l0, W_r0, b_l0.reshape(1, D))
    h = _make_spmm(relu=True)(yl, yr, zl, zr, src, dst, zrow, zcnt, ones)
    yl1, yr1, zl1, zr1 = _make_prep(split=True)(
        h[:, :DH], h[:, DH:], W_l1, W_r1, b_l1.reshape(1, D))
    out = _make_spmm(relu=False)(yl1, yr1, zl1, zr1, src, dst, zrow, zcnt, ones)
    return out


# speedup vs baseline: 4.1607x; 4.1607x over previous
import functools

import jax
import jax.numpy as jnp
from jax import lax
from jax.experimental import pallas as pl
from jax.experimental.pallas import tpu as pltpu
from jax.experimental.pallas import tpu_sc as plsc

N = 10000
E = 160000
D = 256
DH = D // 2
NPAD = 10240
ROWS_PER_TILE = 640
SUB = 80
CHUNK = 128
NCHUNKS = E // CHUNK
NBLK = 10
BLK = N // NBLK


def _prep_compute(x, wl_ref, wr_ref, b_ref, yl_ref, yr_ref, zl_ref, zr_ref):
    y = jnp.dot(x, wl_ref[...], preferred_element_type=jnp.float32)
    z = jnp.dot(x, wr_ref[...], preferred_element_type=jnp.float32) + b_ref[...]
    yl_ref[...] = y[:, :DH]
    yr_ref[...] = y[:, DH:]
    zl_ref[...] = z[:, :DH]
    zr_ref[...] = z[:, DH:]


def _prep_body_full(x_ref, wl_ref, wr_ref, b_ref, *out_refs):
    _prep_compute(x_ref[...], wl_ref, wr_ref, b_ref, *out_refs)


def _prep_body_split(x0_ref, x1_ref, wl_ref, wr_ref, b_ref, *out_refs):
    x = jnp.concatenate([x0_ref[...], x1_ref[...]], axis=1)
    _prep_compute(x, wl_ref, wr_ref, b_ref, *out_refs)


def _make_prep(split):
    if split:
        body = _prep_body_split
        x_specs = [pl.BlockSpec((BLK, DH), lambda i: (i, 0)),
                   pl.BlockSpec((BLK, DH), lambda i: (i, 0))]
    else:
        body = _prep_body_full
        x_specs = [pl.BlockSpec((BLK, D), lambda i: (i, 0))]
    half = pl.BlockSpec((BLK, DH), lambda i: (i, 0))
    return pl.pallas_call(
        body,
        grid=(NBLK,),
        in_specs=x_specs + [
            pl.BlockSpec((D, D), lambda i: (0, 0)),
            pl.BlockSpec((D, D), lambda i: (0, 0)),
            pl.BlockSpec((1, D), lambda i: (0, 0)),
        ],
        out_specs=[half, half, half, half],
        out_shape=[jax.ShapeDtypeStruct((N, DH), jnp.float32)] * 4,
    )


def _spmm_body(relu, yl, yr, zl, zr, src_hbm, dst_hbm, zrow, zcnt, ones_hbm,
               out_hbm, src_v, dst_v, rows_v, ones_v, abuf, zbuf, outb, cbuf,
               acc, cnt, sem):
    c = lax.axis_index("c")
    s = lax.axis_index("s")
    row0 = s * ROWS_PER_TILE
    nrows = jnp.minimum(ROWS_PER_TILE, N - row0)
    nsub = nrows // SUB

    def zero_j(j, carry):
        pltpu.sync_copy(zrow, acc.at[pl.ds(row0 + j * SUB, SUB)])
        return carry
    lax.fori_loop(0, nsub, zero_j, 0)

    @pl.when(s == 0)
    def _():
        pltpu.sync_copy(zcnt, cnt)

    pltpu.sync_copy(ones_hbm, ones_v)
    plsc.subcore_barrier()

    nk = (NCHUNKS - s + 15) // 16

    def edge_k(k, carry):
        base = (k * 16 + s) * CHUNK
        pltpu.sync_copy(src_hbm.at[pl.ds(base, CHUNK)], src_v)
        pltpu.sync_copy(dst_hbm.at[pl.ds(base, CHUNK)], dst_v)

        @pl.when(c == 0)
        def _():
            pltpu.async_copy(yl.at[src_v], rows_v, sem).wait()

        @pl.when(c == 1)
        def _():
            pltpu.async_copy(yr.at[src_v], rows_v, sem).wait()

        pltpu.sync_copy(rows_v, acc.at[dst_v], add=True)
        pltpu.sync_copy(ones_v, cnt.at[dst_v], add=True)
        return carry
    lax.fori_loop(0, nk, edge_k, 0)
    plsc.subcore_barrier()

    def out_j(j, carry):
        base = row0 + j * SUB
        pltpu.sync_copy(acc.at[pl.ds(base, SUB)], abuf)
        pltpu.sync_copy(cnt.at[pl.ds(base, SUB)], cbuf.at[pl.ds(0, SUB)])

        @pl.when(c == 0)
        def _():
            pltpu.sync_copy(zl.at[pl.ds(base, SUB)], zbuf)

        @pl.when(c == 1)
        def _():
            pltpu.sync_copy(zr.at[pl.ds(base, SUB)], zbuf)

        def row_r(r, carry2):
            cv = jnp.broadcast_to(cbuf[pl.ds(r, 16)][0], (16,))
            inv = 1.0 / jnp.maximum(cv, 1.0)

            def vec_v(v, carry3):
                val = abuf[r, pl.ds(v * 16, 16)] * inv + zbuf[r, pl.ds(v * 16, 16)]
                if relu:
                    val = jnp.maximum(val, 0.0)
                outb[r, pl.ds(v * 16, 16)] = val
                return carry3
            lax.fori_loop(0, DH // 16, vec_v, 0)
            return carry2
        lax.fori_loop(0, SUB, row_r, 0)
        pltpu.sync_copy(outb, out_hbm.at[pl.ds(base, SUB), pl.ds(c * DH, DH)])
        return carry
    lax.fori_loop(0, nsub, out_j, 0)


def _make_spmm(relu):
    mesh = plsc.VectorSubcoreMesh(core_axis_name="c", subcore_axis_name="s")
    return pl.kernel(
        functools.partial(_spmm_body, relu),
        out_type=jax.ShapeDtypeStruct((N, D), jnp.float32),
        mesh=mesh,
        scratch_types=[
            pltpu.VMEM((CHUNK,), jnp.int32),
            pltpu.VMEM((CHUNK,), jnp.int32),
            pltpu.VMEM((CHUNK, DH), jnp.float32),
            pltpu.VMEM((CHUNK,), jnp.float32),
            pltpu.VMEM((SUB, DH), jnp.float32),
            pltpu.VMEM((SUB, DH), jnp.float32),
            pltpu.VMEM((SUB, DH), jnp.float32),
            pltpu.VMEM((SUB + 16,), jnp.float32),
            pltpu.VMEM_SHARED((NPAD, DH), jnp.float32),
            pltpu.VMEM_SHARED((NPAD,), jnp.float32),
            pltpu.SemaphoreType.DMA,
        ],
    )


def kernel(x, edge_index, W_l0, b_l0, W_r0, W_l1, b_l1, W_r1):
    src = edge_index[0]
    dst = edge_index[1]
    zrow = jnp.zeros((SUB, DH), jnp.float32)
    zcnt = jnp.zeros((NPAD,), jnp.float32)
    ones = jnp.ones((CHUNK,), jnp.float32)

    yl, yr, zl, zr = _make_prep(split=False)(x, W_l0, W_r0, b_l0.reshape(1, D))
    h = _make_spmm(relu=True)(yl, yr, zl, zr, src, dst, zrow, zcnt, ones)
    yl1, yr1, zl1, zr1 = _make_prep(split=True)(
        h[:, :DH], h[:, DH:], W_l1, W_r1, b_l1.reshape(1, D))
    out = _make_spmm(relu=False)(yl1, yr1, zl1, zr1, src, dst, zrow, zcnt, ones)
    return out


# speedup vs baseline: 7.4940x; 1.8011x over previous
import functools

import jax
import jax.numpy as jnp
from jax import lax
from jax.experimental import pallas as pl
from jax.experimental.pallas import tpu as pltpu
from jax.experimental.pallas import tpu_sc as plsc

N = 10000
E = 160000
D = 256
DH = D // 2
ROWS_PER_TILE = 640
SUB = 80
CHUNK = 128
NCHUNKS = E // CHUNK
NKP = 80
EPAD = 16 * NKP * CHUNK
NBLK = 10
BLK = N // NBLK


def _prep_compute(x, wl_ref, wr_ref, b_ref, yl_ref, yr_ref, zl_ref, zr_ref):
    y = jnp.dot(x, wl_ref[...], preferred_element_type=jnp.float32)
    z = jnp.dot(x, wr_ref[...], preferred_element_type=jnp.float32) + b_ref[...]
    yl_ref[...] = y[:, :DH]
    yr_ref[...] = y[:, DH:]
    zl_ref[...] = z[:, :DH]
    zr_ref[...] = z[:, DH:]


def _prep_body_full(x_ref, wl_ref, wr_ref, b_ref, *out_refs):
    _prep_compute(x_ref[...], wl_ref, wr_ref, b_ref, *out_refs)


def _prep_body_split(x0_ref, x1_ref, wl_ref, wr_ref, b_ref, *out_refs):
    x = jnp.concatenate([x0_ref[...], x1_ref[...]], axis=1)
    _prep_compute(x, wl_ref, wr_ref, b_ref, *out_refs)


def _make_prep(split):
    if split:
        body = _prep_body_split
        x_specs = [pl.BlockSpec((BLK, DH), lambda i: (i, 0)),
                   pl.BlockSpec((BLK, DH), lambda i: (i, 0))]
    else:
        body = _prep_body_full
        x_specs = [pl.BlockSpec((BLK, D), lambda i: (i, 0))]
    half = pl.BlockSpec((BLK, DH), lambda i: (i, 0))
    return pl.pallas_call(
        body,
        grid=(NBLK,),
        in_specs=x_specs + [
            pl.BlockSpec((D, D), lambda i: (0, 0)),
            pl.BlockSpec((D, D), lambda i: (0, 0)),
            pl.BlockSpec((1, D), lambda i: (0, 0)),
        ],
        out_specs=[half, half, half, half],
        out_shape=[jax.ShapeDtypeStruct((N, DH), jnp.float32)] * 4,
    )


def _spmm_body(relu, yl, yr, zl, zr, src_hbm, dst_hbm, zrow, zcnt, ones_hbm,
               out_hbm, src_v, dring, buf0, buf1, ones_v, cbuf, acc, cnt,
               semg0, semg1, semi0, semi1):
    c = lax.axis_index("c")
    s = lax.axis_index("s")
    row0 = s * ROWS_PER_TILE
    nrows = jnp.minimum(ROWS_PER_TILE, N - row0)
    nsub = nrows // SUB

    def zero_j(j, carry):
        pltpu.sync_copy(zrow, acc.at[pl.ds(row0 + j * SUB, SUB)])
        return carry
    lax.fori_loop(0, nsub, zero_j, 0)

    @pl.when(s == 0)
    def _():
        pltpu.sync_copy(zcnt, cnt)

    pltpu.sync_copy(ones_hbm, ones_v)
    plsc.subcore_barrier()

    start = s * NKP
    nk = jnp.clip(NCHUNKS - start, 0, NKP)
    pltpu.sync_copy(src_hbm.at[pl.ds(start * CHUNK, NKP * CHUNK)], src_v)

    bufs = (buf0, buf1)
    semg = (semg0, semg1)
    semi = (semi0, semi1)

    def fire(k, b):
        pltpu.async_copy(dst_hbm.at[start + k], dring.at[b], semi[b])

        @pl.when(c == 0)
        def _():
            pltpu.async_copy(yl.at[src_v.at[pl.ds(k * CHUNK, CHUNK)]],
                             bufs[b], semg[b])

        @pl.when(c == 1)
        def _():
            pltpu.async_copy(yr.at[src_v.at[pl.ds(k * CHUNK, CHUNK)]],
                             bufs[b], semg[b])

    for b in range(2):
        @pl.when(b < nk)
        def _(b=b):
            fire(b, b)

    def pair_k(k2, carry):
        for b in range(2):
            k = k2 * 2 + b

            @pl.when(k < nk)
            def _(k=k, b=b):
                pltpu.make_async_copy(dst_hbm.at[0], dring.at[b], semi[b]).wait()
                pltpu.make_async_copy(yl.at[src_v.at[pl.ds(0, CHUNK)]],
                                      bufs[b], semg[b]).wait()
                pltpu.sync_copy(bufs[b], acc.at[dring.at[b]], add=True)
                pltpu.sync_copy(ones_v, cnt.at[dring.at[b]], add=True)

                @pl.when(k + 2 < nk)
                def _():
                    fire(k + 2, b)
        return carry
    lax.fori_loop(0, NKP // 2, pair_k, 0)
    plsc.subcore_barrier()

    abuf = buf0.at[pl.ds(0, SUB)]
    zbuf = buf1.at[pl.ds(0, SUB)]

    def out_j(j, carry):
        base = row0 + j * SUB
        pltpu.sync_copy(acc.at[pl.ds(base, SUB)], abuf)
        pltpu.sync_copy(cnt.at[pl.ds(base, SUB)], cbuf.at[pl.ds(0, SUB)])

        @pl.when(c == 0)
        def _():
            pltpu.sync_copy(zl.at[pl.ds(base, SUB)], zbuf)

        @pl.when(c == 1)
        def _():
            pltpu.sync_copy(zr.at[pl.ds(base, SUB)], zbuf)

        def row_r(r, carry2):
            cv = jnp.broadcast_to(cbuf[pl.ds(r, 16)][0], (16,))
            inv = 1.0 / jnp.maximum(cv, 1.0)

            def vec_v(v, carry3):
                val = buf0[r, pl.ds(v * 16, 16)] * inv + buf1[r, pl.ds(v * 16, 16)]
                if relu:
                    val = jnp.maximum(val, 0.0)
                buf0[r, pl.ds(v * 16, 16)] = val
                return carry3
            lax.fori_loop(0, DH // 16, vec_v, 0)
            return carry2
        lax.fori_loop(0, SUB, row_r, 0)
        pltpu.sync_copy(abuf, out_hbm.at[pl.ds(base, SUB), pl.ds(c * DH, DH)])
        return carry
    lax.fori_loop(0, nsub, out_j, 0)


def _make_spmm(relu):
    mesh = plsc.VectorSubcoreMesh(core_axis_name="c", subcore_axis_name="s")
    return pl.kernel(
        functools.partial(_spmm_body, relu),
        out_type=jax.ShapeDtypeStruct((N, D), jnp.float32),
        mesh=mesh,
        scratch_types=[
            pltpu.VMEM((NKP * CHUNK,), jnp.int32),
            pltpu.VMEM((2, CHUNK), jnp.int32),
            pltpu.VMEM((CHUNK, DH), jnp.float32),
            pltpu.VMEM((CHUNK, DH), jnp.float32),
            pltpu.VMEM((CHUNK,), jnp.float32),
            pltpu.VMEM((SUB + 16,), jnp.float32),
            pltpu.VMEM_SHARED((N, DH), jnp.float32),
            pltpu.VMEM_SHARED((N,), jnp.float32),
            pltpu.SemaphoreType.DMA,
            pltpu.SemaphoreType.DMA,
            pltpu.SemaphoreType.DMA,
            pltpu.SemaphoreType.DMA,
        ],
    )


def kernel(x, edge_index, W_l0, b_l0, W_r0, W_l1, b_l1, W_r1):
    pad1 = jnp.zeros((EPAD - E,), jnp.int32)
    pad2 = jnp.zeros((16 * NKP - NCHUNKS, CHUNK), jnp.int32)
    src = jnp.concatenate([edge_index[0], pad1])
    dst = jnp.concatenate([edge_index[1].reshape(NCHUNKS, CHUNK), pad2])
    zrow = jnp.zeros((SUB, DH), jnp.float32)
    zcnt = jnp.zeros((N,), jnp.float32)
    ones = jnp.ones((CHUNK,), jnp.float32)

    yl, yr, zl, zr = _make_prep(split=False)(x, W_l0, W_r0, b_l0.reshape(1, D))
    h = _make_spmm(relu=True)(yl, yr, zl, zr, src, dst, zrow, zcnt, ones)
    yl1, yr1, zl1, zr1 = _make_prep(split=True)(
        h[:, :DH], h[:, DH:], W_l1, W_r1, b_l1.reshape(1, D))
    out = _make_spmm(relu=False)(yl1, yr1, zl1, zr1, src, dst, zrow, zcnt, ones)
    return out


# speedup vs baseline: 8.3160x; 1.1097x over previous
import functools

import jax
import jax.numpy as jnp
from jax import lax
from jax.experimental import pallas as pl
from jax.experimental.pallas import tpu as pltpu
from jax.experimental.pallas import tpu_sc as plsc

N = 10000
E = 160000
D = 256
DH = D // 2
ROWS_PER_TILE = 640
SUB = 80
CHUNK = 128
NCHUNKS = E // CHUNK
NKP = 80
EPAD = 16 * NKP * CHUNK
NBLK = 10
BLK = N // NBLK


def _prep_compute(x, wl_ref, wr_ref, b_ref, yl_ref, yr_ref, zl_ref, zr_ref):
    y = jnp.dot(x, wl_ref[...], preferred_element_type=jnp.float32)
    z = jnp.dot(x, wr_ref[...], preferred_element_type=jnp.float32) + b_ref[...]
    yl_ref[...] = y[:, :DH]
    yr_ref[...] = y[:, DH:]
    zl_ref[...] = z[:, :DH]
    zr_ref[...] = z[:, DH:]


def _prep_body_full(x_ref, wl_ref, wr_ref, b_ref, *out_refs):
    _prep_compute(x_ref[...], wl_ref, wr_ref, b_ref, *out_refs)


def _make_prep():
    half = pl.BlockSpec((BLK, DH), lambda i: (i, 0))
    return pl.pallas_call(
        _prep_body_full,
        grid=(NBLK,),
        in_specs=[
            pl.BlockSpec((BLK, D), lambda i: (i, 0)),
            pl.BlockSpec((D, D), lambda i: (0, 0)),
            pl.BlockSpec((D, D), lambda i: (0, 0)),
            pl.BlockSpec((1, D), lambda i: (0, 0)),
        ],
        out_specs=[half, half, half, half],
        out_shape=[jax.ShapeDtypeStruct((N, DH), jnp.float32)] * 4,
    )


def _spmm_body(relu, yl, yr, zl, zr, src_hbm, dst_hbm, zrow, zcnt, ones_hbm,
               out_hbm, src_v, dring, buf0, buf1, ones_v, cbuf, acc, cnt,
               semg0, semg1, semi0, semi1):
    c = lax.axis_index("c")
    s = lax.axis_index("s")
    row0 = s * ROWS_PER_TILE
    nrows = jnp.minimum(ROWS_PER_TILE, N - row0)
    nsub = nrows // SUB

    @pl.when(s < 15)
    def _():
        pltpu.sync_copy(zrow, acc.at[pl.ds(row0, ROWS_PER_TILE)])

    @pl.when(s == 15)
    def _():
        pltpu.sync_copy(zrow.at[pl.ds(0, N - 15 * ROWS_PER_TILE)],
                        acc.at[pl.ds(15 * ROWS_PER_TILE,
                                     N - 15 * ROWS_PER_TILE)])

    @pl.when(s == 0)
    def _():
        pltpu.sync_copy(zcnt, cnt)

    pltpu.sync_copy(ones_hbm, ones_v)
    plsc.subcore_barrier()

    start = s * NKP
    nk = jnp.clip(NCHUNKS - start, 0, NKP)
    pltpu.sync_copy(src_hbm.at[pl.ds(start * CHUNK, NKP * CHUNK)], src_v)

    bufs = (buf0, buf1)
    semg = (semg0, semg1)
    semi = (semi0, semi1)

    def fire(k, b):
        pltpu.async_copy(dst_hbm.at[start + k], dring.at[b], semi[b])

        @pl.when(c == 0)
        def _():
            pltpu.async_copy(yl.at[src_v.at[pl.ds(k * CHUNK, CHUNK)]],
                             bufs[b], semg[b])

        @pl.when(c == 1)
        def _():
            pltpu.async_copy(yr.at[src_v.at[pl.ds(k * CHUNK, CHUNK)]],
                             bufs[b], semg[b])

    for b in range(2):
        @pl.when(b < nk)
        def _(b=b):
            fire(b, b)

    def pair_k(k2, carry):
        for b in range(2):
            k = k2 * 2 + b

            @pl.when(k < nk)
            def _(k=k, b=b):
                pltpu.make_async_copy(dst_hbm.at[0], dring.at[b], semi[b]).wait()
                pltpu.make_async_copy(yl.at[src_v.at[pl.ds(0, CHUNK)]],
                                      bufs[b], semg[b]).wait()
                pltpu.sync_copy(bufs[b], acc.at[dring.at[b]], add=True)
                pltpu.sync_copy(ones_v, cnt.at[dring.at[b]], add=True)

                @pl.when(k + 2 < nk)
                def _():
                    fire(k + 2, b)
        return carry
    lax.fori_loop(0, NKP // 2, pair_k, 0)
    plsc.subcore_barrier()

    abuf = buf0.at[pl.ds(0, SUB)]
    zbuf = buf1.at[pl.ds(0, SUB)]
    cslc = cbuf.at[pl.ds(0, SUB)]

    def out_j(j, carry):
        base = row0 + j * SUB
        @pl.when(c == 0)
        def _():
            pltpu.async_copy(zl.at[pl.ds(base, SUB)], zbuf, semg1)

        @pl.when(c == 1)
        def _():
            pltpu.async_copy(zr.at[pl.ds(base, SUB)], zbuf, semg1)

        pltpu.async_copy(cnt.at[pl.ds(base, SUB)], cslc, semi0)

        @pl.when(j > 0)
        def _():
            pltpu.make_async_copy(
                abuf, out_hbm.at[pl.ds(base, SUB), pl.ds(c * DH, DH)],
                semi1).wait()

        pltpu.async_copy(acc.at[pl.ds(base, SUB)], abuf, semg0)
        pltpu.make_async_copy(acc.at[pl.ds(base, SUB)], abuf, semg0).wait()
        pltpu.make_async_copy(cnt.at[pl.ds(base, SUB)], cslc, semi0).wait()
        pltpu.make_async_copy(zl.at[pl.ds(base, SUB)], zbuf, semg1).wait()

        def grp_g(g, carry2):
            invv = 1.0 / jnp.maximum(cbuf[pl.ds(g * 16, 16)], 1.0)
            for r in range(16):
                iv = jnp.broadcast_to(invv[r], (16,))
                for v in range(DH // 16):
                    sl = pl.ds(v * 16, 16)
                    val = buf0[g * 16 + r, sl] * iv + buf1[g * 16 + r, sl]
                    if relu:
                        val = jnp.maximum(val, 0.0)
                    buf0[g * 16 + r, sl] = val
            return carry2
        lax.fori_loop(0, SUB // 16, grp_g, 0)
        pltpu.async_copy(abuf, out_hbm.at[pl.ds(base, SUB), pl.ds(c * DH, DH)],
                         semi1)
        return carry
    lax.fori_loop(0, nsub, out_j, 0)
    pltpu.make_async_copy(
        abuf, out_hbm.at[pl.ds(row0 + (nsub - 1) * SUB, SUB),
                         pl.ds(c * DH, DH)], semi1).wait()


def _make_spmm(relu):
    mesh = plsc.VectorSubcoreMesh(core_axis_name="c", subcore_axis_name="s")
    return pl.kernel(
        functools.partial(_spmm_body, relu),
        out_type=jax.ShapeDtypeStruct((N, D), jnp.float32),
        mesh=mesh,
        scratch_types=[
            pltpu.VMEM((NKP * CHUNK,), jnp.int32),
            pltpu.VMEM((2, CHUNK), jnp.int32),
            pltpu.VMEM((CHUNK, DH), jnp.float32),
            pltpu.VMEM((CHUNK, DH), jnp.float32),
            pltpu.VMEM((CHUNK,), jnp.float32),
            pltpu.VMEM((SUB + 16,), jnp.float32),
            pltpu.VMEM_SHARED((N, DH), jnp.float32),
            pltpu.VMEM_SHARED((N,), jnp.float32),
            pltpu.SemaphoreType.DMA,
            pltpu.SemaphoreType.DMA,
            pltpu.SemaphoreType.DMA,
            pltpu.SemaphoreType.DMA,
        ],
    )


def kernel(x, edge_index, W_l0, b_l0, W_r0, W_l1, b_l1, W_r1):
    pad1 = jnp.zeros((EPAD - E,), jnp.int32)
    pad2 = jnp.zeros((16 * NKP - NCHUNKS, CHUNK), jnp.int32)
    src = jnp.concatenate([edge_index[0], pad1])
    dst = jnp.concatenate([edge_index[1].reshape(NCHUNKS, CHUNK), pad2])
    zrow = jnp.zeros((ROWS_PER_TILE, DH), jnp.float32)
    zcnt = jnp.zeros((N,), jnp.float32)
    ones = jnp.ones((CHUNK,), jnp.float32)

    yl, yr, zl, zr = _make_prep()(x, W_l0, W_r0, b_l0.reshape(1, D))
    h = _make_spmm(relu=True)(yl, yr, zl, zr, src, dst, zrow, zcnt, ones)
    yl1, yr1, zl1, zr1 = _make_prep()(h, W_l1, W_r1, b_l1.reshape(1, D))
    out = _make_spmm(relu=False)(yl1, yr1, zl1, zr1, src, dst, zrow, zcnt, ones)
    return out


# speedup vs baseline: 8.3305x; 1.0017x over previous
import functools

import jax
import jax.numpy as jnp
from jax import lax
from jax.experimental import pallas as pl
from jax.experimental.pallas import tpu as pltpu
from jax.experimental.pallas import tpu_sc as plsc

N = 10000
E = 160000
D = 256
DH = D // 2
ROWS_PER_TILE = 640
SUB = 80
CHUNK = 128
NCHUNKS = E // CHUNK
NKP = 80
NBLK = 10
BLK = N // NBLK


def _prep_compute(x, wl_ref, wr_ref, b_ref, yl_ref, yr_ref, zl_ref, zr_ref):
    y = jnp.dot(x, wl_ref[...], preferred_element_type=jnp.float32)
    z = jnp.dot(x, wr_ref[...], preferred_element_type=jnp.float32) + b_ref[...]
    yl_ref[...] = y[:, :DH]
    yr_ref[...] = y[:, DH:]
    zl_ref[...] = z[:, :DH]
    zr_ref[...] = z[:, DH:]


def _prep_body_full(x_ref, wl_ref, wr_ref, b_ref, *out_refs):
    _prep_compute(x_ref[...], wl_ref, wr_ref, b_ref, *out_refs)


def _make_prep():
    half = pl.BlockSpec((BLK, DH), lambda i: (i, 0))
    return pl.pallas_call(
        _prep_body_full,
        grid=(NBLK,),
        in_specs=[
            pl.BlockSpec((BLK, D), lambda i: (i, 0)),
            pl.BlockSpec((D, D), lambda i: (0, 0)),
            pl.BlockSpec((D, D), lambda i: (0, 0)),
            pl.BlockSpec((1, D), lambda i: (0, 0)),
        ],
        out_specs=[half, half, half, half],
        out_shape=[jax.ShapeDtypeStruct((N, DH), jnp.float32)] * 4,
    )


def _spmm_body(relu, yl, yr, zl, zr, src_hbm, dst_hbm, zrow, zcnt, ones_hbm,
               out_hbm, src_v, dring, buf0, buf1, ones_v, cbuf, acc, cnt,
               semg0, semg1, semi0, semi1):
    c = lax.axis_index("c")
    s = lax.axis_index("s")
    row0 = s * ROWS_PER_TILE
    nrows = jnp.minimum(ROWS_PER_TILE, N - row0)
    nsub = nrows // SUB

    @pl.when(s < 15)
    def _():
        pltpu.sync_copy(zrow, acc.at[pl.ds(row0, ROWS_PER_TILE)])

    @pl.when(s == 15)
    def _():
        pltpu.sync_copy(zrow.at[pl.ds(0, N - 15 * ROWS_PER_TILE)],
                        acc.at[pl.ds(15 * ROWS_PER_TILE,
                                     N - 15 * ROWS_PER_TILE)])

    @pl.when(s == 0)
    def _():
        pltpu.sync_copy(zcnt, cnt)

    pltpu.sync_copy(ones_hbm, ones_v)
    plsc.subcore_barrier()

    start = jnp.minimum(s * NKP, NCHUNKS - NKP)
    klo = jnp.maximum(s * NKP - (NCHUNKS - NKP), 0)
    pltpu.sync_copy(src_hbm.at[pl.ds(start * CHUNK, NKP * CHUNK)], src_v)

    bufs = (buf0, buf1)
    semg = (semg0, semg1)
    semi = (semi0, semi1)

    def fire(k, b):
        pltpu.async_copy(dst_hbm.at[start + k], dring.at[b], semi[b])

        @pl.when(c == 0)
        def _():
            pltpu.async_copy(yl.at[src_v.at[pl.ds(k * CHUNK, CHUNK)]],
                             bufs[b], semg[b])

        @pl.when(c == 1)
        def _():
            pltpu.async_copy(yr.at[src_v.at[pl.ds(k * CHUNK, CHUNK)]],
                             bufs[b], semg[b])

    for b in range(2):
        @pl.when(klo + b < NKP)
        def _(b=b):
            fire(klo + b, b)

    def pair_k(k2, carry):
        for b in range(2):
            k = klo + k2 * 2 + b

            @pl.when(k < NKP)
            def _(k=k, b=b):
                pltpu.make_async_copy(dst_hbm.at[0], dring.at[b], semi[b]).wait()
                pltpu.make_async_copy(yl.at[src_v.at[pl.ds(0, CHUNK)]],
                                      bufs[b], semg[b]).wait()
                pltpu.sync_copy(bufs[b], acc.at[dring.at[b]], add=True)
                pltpu.sync_copy(ones_v, cnt.at[dring.at[b]], add=True)

                @pl.when(k + 2 < NKP)
                def _():
                    fire(k + 2, b)
        return carry
    lax.fori_loop(0, NKP // 2, pair_k, 0)
    plsc.subcore_barrier()

    abuf = buf0.at[pl.ds(0, SUB)]
    zbuf = buf1.at[pl.ds(0, SUB)]
    cslc = cbuf.at[pl.ds(0, SUB)]

    def out_j(j, carry):
        base = row0 + j * SUB
        @pl.when(c == 0)
        def _():
            pltpu.async_copy(zl.at[pl.ds(base, SUB)], zbuf, semg1)

        @pl.when(c == 1)
        def _():
            pltpu.async_copy(zr.at[pl.ds(base, SUB)], zbuf, semg1)

        pltpu.async_copy(cnt.at[pl.ds(base, SUB)], cslc, semi0)

        @pl.when(j > 0)
        def _():
            pltpu.make_async_copy(
                abuf, out_hbm.at[pl.ds(base, SUB), pl.ds(c * DH, DH)],
                semi1).wait()

        pltpu.async_copy(acc.at[pl.ds(base, SUB)], abuf, semg0)
        pltpu.make_async_copy(acc.at[pl.ds(base, SUB)], abuf, semg0).wait()
        pltpu.make_async_copy(cnt.at[pl.ds(base, SUB)], cslc, semi0).wait()
        pltpu.make_async_copy(zl.at[pl.ds(base, SUB)], zbuf, semg1).wait()

        def grp_g(g, carry2):
            invv = 1.0 / jnp.maximum(cbuf[pl.ds(g * 16, 16)], 1.0)
            for r in range(16):
                iv = jnp.broadcast_to(invv[r], (16,))
                for v in range(DH // 16):
                    sl = pl.ds(v * 16, 16)
                    val = buf0[g * 16 + r, sl] * iv + buf1[g * 16 + r, sl]
                    if relu:
                        val = jnp.maximum(val, 0.0)
                    buf0[g * 16 + r, sl] = val
            return carry2
        lax.fori_loop(0, SUB // 16, grp_g, 0)
        pltpu.async_copy(abuf, out_hbm.at[pl.ds(base, SUB), pl.ds(c * DH, DH)],
                         semi1)
        return carry
    lax.fori_loop(0, nsub, out_j, 0)
    pltpu.make_async_copy(
        abuf, out_hbm.at[pl.ds(row0 + (nsub - 1) * SUB, SUB),
                         pl.ds(c * DH, DH)], semi1).wait()


def _make_spmm(relu):
    mesh = plsc.VectorSubcoreMesh(core_axis_name="c", subcore_axis_name="s")
    return pl.kernel(
        functools.partial(_spmm_body, relu),
        out_type=jax.ShapeDtypeStruct((N, D), jnp.float32),
        mesh=mesh,
        scratch_types=[
            pltpu.VMEM((NKP * CHUNK,), jnp.int32),
            pltpu.VMEM((2, CHUNK), jnp.int32),
            pltpu.VMEM((CHUNK, DH), jnp.float32),
            pltpu.VMEM((CHUNK, DH), jnp.float32),
            pltpu.VMEM((CHUNK,), jnp.float32),
            pltpu.VMEM((SUB + 16,), jnp.float32),
            pltpu.VMEM_SHARED((N, DH), jnp.float32),
            pltpu.VMEM_SHARED((N,), jnp.float32),
            pltpu.SemaphoreType.DMA,
            pltpu.SemaphoreType.DMA,
            pltpu.SemaphoreType.DMA,
            pltpu.SemaphoreType.DMA,
        ],
    )


def kernel(x, edge_index, W_l0, b_l0, W_r0, W_l1, b_l1, W_r1):
    src = edge_index[0]
    dst = edge_index[1].reshape(NCHUNKS, CHUNK)
    zrow = jnp.zeros((ROWS_PER_TILE, DH), jnp.float32)
    zcnt = jnp.zeros((N,), jnp.float32)
    ones = jnp.ones((CHUNK,), jnp.float32)

    yl, yr, zl, zr = _make_prep()(x, W_l0, W_r0, b_l0.reshape(1, D))
    h = _make_spmm(relu=True)(yl, yr, zl, zr, src, dst, zrow, zcnt, ones)
    yl1, yr1, zl1, zr1 = _make_prep()(h, W_l1, W_r1, b_l1.reshape(1, D))
    out = _make_spmm(relu=False)(yl1, yr1, zl1, zr1, src, dst, zrow, zcnt, ones)
    return out


# speedup vs baseline: 8.4689x; 1.0166x over previous
import functools

import jax
import jax.numpy as jnp
from jax import lax
from jax.experimental import pallas as pl
from jax.experimental.pallas import tpu as pltpu
from jax.experimental.pallas import tpu_sc as plsc

N = 10000
E = 160000
D = 256
DH = D // 2
ROWS_PER_TILE = 640
SUB = 80
CHUNK = 128
NCHUNKS = E // CHUNK
NKP = 80
NBLK = 10
BLK = N // NBLK


def _prep_compute(x, wl_ref, wr_ref, b_ref, yl_ref, yr_ref, zl_ref, zr_ref):
    y = jnp.dot(x, wl_ref[...], preferred_element_type=jnp.float32)
    z = jnp.dot(x, wr_ref[...], preferred_element_type=jnp.float32) + b_ref[...]
    yl_ref[...] = y[:, :DH]
    yr_ref[...] = y[:, DH:]
    zl_ref[...] = z[:, :DH]
    zr_ref[...] = z[:, DH:]


def _prep_body_full(x_ref, wl_ref, wr_ref, b_ref, *out_refs):
    _prep_compute(x_ref[...], wl_ref, wr_ref, b_ref, *out_refs)


def _make_prep():
    half = pl.BlockSpec((BLK, DH), lambda i: (i, 0))
    return pl.pallas_call(
        _prep_body_full,
        grid=(NBLK,),
        in_specs=[
            pl.BlockSpec((BLK, D), lambda i: (i, 0)),
            pl.BlockSpec((D, D), lambda i: (0, 0)),
            pl.BlockSpec((D, D), lambda i: (0, 0)),
            pl.BlockSpec((1, D), lambda i: (0, 0)),
        ],
        out_specs=[half, half, half, half],
        out_shape=[jax.ShapeDtypeStruct((N, DH), jnp.float32)] * 4,
    )


def _spmm_body(do_counts, refs):
    relu = do_counts
    if do_counts:
        (yl, yr, zl, zr, src_hbm, dst_hbm, zrow, zcnt, ones_hbm,
         out_hbm, cnt_out, src_v, dring, buf0, buf1, ones_v, cbuf, acc, cnt,
         semg0, semg1, semi0, semi1) = refs
    else:
        (yl, yr, zl, zr, src_hbm, dst_hbm, zrow, cnt_hbm,
         out_hbm, src_v, dring, buf0, buf1, cbuf, acc,
         semg0, semg1, semi0, semi1) = refs
    c = lax.axis_index("c")
    s = lax.axis_index("s")
    row0 = s * ROWS_PER_TILE
    nrows = jnp.minimum(ROWS_PER_TILE, N - row0)
    nsub = nrows // SUB

    @pl.when(s < 15)
    def _():
        pltpu.sync_copy(zrow, acc.at[pl.ds(row0, ROWS_PER_TILE)])

    @pl.when(s == 15)
    def _():
        pltpu.sync_copy(zrow.at[pl.ds(0, N - 15 * ROWS_PER_TILE)],
                        acc.at[pl.ds(15 * ROWS_PER_TILE,
                                     N - 15 * ROWS_PER_TILE)])

    if do_counts:
        @pl.when(s == 0)
        def _():
            pltpu.sync_copy(zcnt, cnt)

        pltpu.sync_copy(ones_hbm, ones_v)
    plsc.subcore_barrier()

    start = jnp.minimum(s * NKP, NCHUNKS - NKP)
    klo = jnp.maximum(s * NKP - (NCHUNKS - NKP), 0)
    pltpu.sync_copy(src_hbm.at[pl.ds(start * CHUNK, NKP * CHUNK)], src_v)

    bufs = (buf0, buf1)
    semg = (semg0, semg1)
    semi = (semi0, semi1)

    def fire(k, b):
        pltpu.async_copy(dst_hbm.at[start + k], dring.at[b], semi[b])

        @pl.when(c == 0)
        def _():
            pltpu.async_copy(yl.at[src_v.at[pl.ds(k * CHUNK, CHUNK)]],
                             bufs[b], semg[b])

        @pl.when(c == 1)
        def _():
            pltpu.async_copy(yr.at[src_v.at[pl.ds(k * CHUNK, CHUNK)]],
                             bufs[b], semg[b])

    for b in range(2):
        @pl.when(klo + b < NKP)
        def _(b=b):
            fire(klo + b, b)

    def pair_k(k2, carry):
        for b in range(2):
            k = klo + k2 * 2 + b

            @pl.when(k < NKP)
            def _(k=k, b=b):
                pltpu.make_async_copy(dst_hbm.at[0], dring.at[b], semi[b]).wait()
                pltpu.make_async_copy(yl.at[src_v.at[pl.ds(0, CHUNK)]],
                                      bufs[b], semg[b]).wait()
                pltpu.sync_copy(bufs[b], acc.at[dring.at[b]], add=True)
                if do_counts:
                    pltpu.sync_copy(ones_v, cnt.at[dring.at[b]], add=True)

                @pl.when(k + 2 < NKP)
                def _():
                    fire(k + 2, b)
        return carry
    lax.fori_loop(0, NKP // 2, pair_k, 0)
    plsc.subcore_barrier()

    if do_counts:
        @pl.when(jnp.logical_and(s == 0, c == 0))
        def _():
            pltpu.sync_copy(cnt, cnt_out)

    abuf = buf0.at[pl.ds(0, SUB)]
    zbuf = buf1.at[pl.ds(0, SUB)]
    cslc = cbuf.at[pl.ds(0, SUB)]

    def out_j(j, carry):
        base = row0 + j * SUB
        @pl.when(c == 0)
        def _():
            pltpu.async_copy(zl.at[pl.ds(base, SUB)], zbuf, semg1)

        @pl.when(c == 1)
        def _():
            pltpu.async_copy(zr.at[pl.ds(base, SUB)], zbuf, semg1)

        cnt_src = cnt if do_counts else cnt_hbm
        pltpu.async_copy(cnt_src.at[pl.ds(base, SUB)], cslc, semi0)

        @pl.when(j > 0)
        def _():
            pltpu.make_async_copy(
                abuf, out_hbm.at[pl.ds(base, SUB), pl.ds(c * DH, DH)],
                semi1).wait()

        pltpu.async_copy(acc.at[pl.ds(base, SUB)], abuf, semg0)
        pltpu.make_async_copy(acc.at[pl.ds(base, SUB)], abuf, semg0).wait()
        pltpu.make_async_copy(cnt_src.at[pl.ds(base, SUB)], cslc, semi0).wait()
        pltpu.make_async_copy(zl.at[pl.ds(base, SUB)], zbuf, semg1).wait()

        def grp_g(g, carry2):
            invv = 1.0 / jnp.maximum(cbuf[pl.ds(g * 16, 16)], 1.0)
            for r in range(16):
                iv = jnp.broadcast_to(invv[r], (16,))
                for v in range(DH // 16):
                    sl = pl.ds(v * 16, 16)
                    val = buf0[g * 16 + r, sl] * iv + buf1[g * 16 + r, sl]
                    if relu:
                        val = jnp.maximum(val, 0.0)
                    buf0[g * 16 + r, sl] = val
            return carry2
        lax.fori_loop(0, SUB // 16, grp_g, 0)
        pltpu.async_copy(abuf, out_hbm.at[pl.ds(base, SUB), pl.ds(c * DH, DH)],
                         semi1)
        return carry
    lax.fori_loop(0, nsub, out_j, 0)
    pltpu.make_async_copy(
        abuf, out_hbm.at[pl.ds(row0 + (nsub - 1) * SUB, SUB),
                         pl.ds(c * DH, DH)], semi1).wait()


def _make_spmm(do_counts):
    mesh = plsc.VectorSubcoreMesh(core_axis_name="c", subcore_axis_name="s")
    scratch = [
        pltpu.VMEM((NKP * CHUNK,), jnp.int32),
        pltpu.VMEM((2, CHUNK), jnp.int32),
        pltpu.VMEM((CHUNK, DH), jnp.float32),
        pltpu.VMEM((CHUNK, DH), jnp.float32),
    ]
    if do_counts:
        scratch.append(pltpu.VMEM((CHUNK,), jnp.float32))
    scratch.append(pltpu.VMEM((SUB + 16,), jnp.float32))
    scratch.append(pltpu.VMEM_SHARED((N, DH), jnp.float32))
    if do_counts:
        scratch.append(pltpu.VMEM_SHARED((N,), jnp.float32))
    scratch += [pltpu.SemaphoreType.DMA] * 4
    if do_counts:
        out_type = (jax.ShapeDtypeStruct((N, D), jnp.float32),
                    jax.ShapeDtypeStruct((N,), jnp.float32))
    else:
        out_type = jax.ShapeDtypeStruct((N, D), jnp.float32)
    return pl.kernel(
        lambda *refs: _spmm_body(do_counts, refs),
        out_type=out_type,
        mesh=mesh,
        scratch_types=scratch,
    )


def kernel(x, edge_index, W_l0, b_l0, W_r0, W_l1, b_l1, W_r1):
    src = edge_index[0]
    dst = edge_index[1].reshape(NCHUNKS, CHUNK)
    zrow = jnp.zeros((ROWS_PER_TILE, DH), jnp.float32)
    zcnt = jnp.zeros((N,), jnp.float32)
    ones = jnp.ones((CHUNK,), jnp.float32)

    yl, yr, zl, zr = _make_prep()(x, W_l0, W_r0, b_l0.reshape(1, D))
    h, cnt = _make_spmm(True)(yl, yr, zl, zr, src, dst, zrow, zcnt, ones)
    yl1, yr1, zl1, zr1 = _make_prep()(h, W_l1, W_r1, b_l1.reshape(1, D))
    out = _make_spmm(False)(yl1, yr1, zl1, zr1, src, dst, zrow, cnt)
    return out


# speedup vs baseline: 8.6006x; 1.0156x over previous
import functools

import jax
import jax.numpy as jnp
from jax import lax
from jax.experimental import pallas as pl
from jax.experimental.pallas import tpu as pltpu
from jax.experimental.pallas import tpu_sc as plsc

N = 10000
E = 160000
D = 256
DH = D // 2
ROWS_PER_TILE = 640
SUB = 80
CHUNK = 128
NCHUNKS = E // CHUNK
NKP = 80
NBLK = 5
BLK = N // NBLK


def _prep_compute(x, wl_ref, wr_ref, b_ref, yl_ref, yr_ref, zl_ref, zr_ref):
    y = jnp.dot(x, wl_ref[...], preferred_element_type=jnp.float32)
    z = jnp.dot(x, wr_ref[...], preferred_element_type=jnp.float32) + b_ref[...]
    yl_ref[...] = y[:, :DH]
    yr_ref[...] = y[:, DH:]
    zl_ref[...] = z[:, :DH]
    zr_ref[...] = z[:, DH:]


def _prep_body_full(x_ref, wl_ref, wr_ref, b_ref, *out_refs):
    _prep_compute(x_ref[...], wl_ref, wr_ref, b_ref, *out_refs)


def _make_prep():
    half = pl.BlockSpec((BLK, DH), lambda i: (i, 0))
    return pl.pallas_call(
        _prep_body_full,
        grid=(NBLK,),
        in_specs=[
            pl.BlockSpec((BLK, D), lambda i: (i, 0)),
            pl.BlockSpec((D, D), lambda i: (0, 0)),
            pl.BlockSpec((D, D), lambda i: (0, 0)),
            pl.BlockSpec((1, D), lambda i: (0, 0)),
        ],
        out_specs=[half, half, half, half],
        out_shape=[jax.ShapeDtypeStruct((N, DH), jnp.float32)] * 4,
    )


def _spmm_body(do_counts, refs):
    relu = do_counts
    if do_counts:
        (yl, yr, zl, zr, src_hbm, dst_hbm, zrow, zcnt, ones_hbm,
         out_hbm, cnt_out, src_v, dring, buf0, buf1, ones_v, cbuf, acc, cnt,
         semg0, semg1, semi0, semi1) = refs
    else:
        (yl, yr, zl, zr, src_hbm, dst_hbm, zrow, cnt_hbm,
         out_hbm, src_v, dring, buf0, buf1, cbuf, acc,
         semg0, semg1, semi0, semi1) = refs
    c = lax.axis_index("c")
    s = lax.axis_index("s")
    row0 = s * ROWS_PER_TILE
    nrows = jnp.minimum(ROWS_PER_TILE, N - row0)
    nsub = nrows // SUB

    @pl.when(s < 15)
    def _():
        pltpu.sync_copy(zrow, acc.at[pl.ds(row0, ROWS_PER_TILE)])

    @pl.when(s == 15)
    def _():
        pltpu.sync_copy(zrow.at[pl.ds(0, N - 15 * ROWS_PER_TILE)],
                        acc.at[pl.ds(15 * ROWS_PER_TILE,
                                     N - 15 * ROWS_PER_TILE)])

    if do_counts:
        @pl.when(s == 0)
        def _():
            pltpu.sync_copy(zcnt, cnt)

        pltpu.sync_copy(ones_hbm, ones_v)
    plsc.subcore_barrier()

    start = jnp.minimum(s * NKP, NCHUNKS - NKP)
    klo = jnp.maximum(s * NKP - (NCHUNKS - NKP), 0)
    pltpu.sync_copy(src_hbm.at[pl.ds(start * CHUNK, NKP * CHUNK)], src_v)

    bufs = (buf0, buf1)
    semg = (semg0, semg1)
    semi = (semi0, semi1)

    def fire(k, b):
        pltpu.async_copy(dst_hbm.at[start + k], dring.at[b], semi[b])

        @pl.when(c == 0)
        def _():
            pltpu.async_copy(yl.at[src_v.at[pl.ds(k * CHUNK, CHUNK)]],
                             bufs[b], semg[b])

        @pl.when(c == 1)
        def _():
            pltpu.async_copy(yr.at[src_v.at[pl.ds(k * CHUNK, CHUNK)]],
                             bufs[b], semg[b])

    for b in range(2):
        @pl.when(klo + b < NKP)
        def _(b=b):
            fire(klo + b, b)

    def pair_k(k2, carry):
        for b in range(2):
            k = klo + k2 * 2 + b

            @pl.when(k < NKP)
            def _(k=k, b=b):
                pltpu.make_async_copy(dst_hbm.at[0], dring.at[b], semi[b]).wait()
                pltpu.make_async_copy(yl.at[src_v.at[pl.ds(0, CHUNK)]],
                                      bufs[b], semg[b]).wait()
                pltpu.sync_copy(bufs[b], acc.at[dring.at[b]], add=True)
                if do_counts:
                    pltpu.sync_copy(ones_v, cnt.at[dring.at[b]], add=True)

                @pl.when(k + 2 < NKP)
                def _():
                    fire(k + 2, b)
        return carry
    lax.fori_loop(0, NKP // 2, pair_k, 0)
    plsc.subcore_barrier()

    if do_counts:
        @pl.when(jnp.logical_and(s == 0, c == 0))
        def _():
            pltpu.sync_copy(cnt, cnt_out)

    abuf = buf0.at[pl.ds(0, SUB)]
    zbuf = buf1.at[pl.ds(0, SUB)]
    cslc = cbuf.at[pl.ds(0, SUB)]

    def out_j(j, carry):
        base = row0 + j * SUB
        @pl.when(c == 0)
        def _():
            pltpu.async_copy(zl.at[pl.ds(base, SUB)], zbuf, semg1)

        @pl.when(c == 1)
        def _():
            pltpu.async_copy(zr.at[pl.ds(base, SUB)], zbuf, semg1)

        cnt_src = cnt if do_counts else cnt_hbm
        pltpu.async_copy(cnt_src.at[pl.ds(base, SUB)], cslc, semi0)

        @pl.when(j > 0)
        def _():
            pltpu.make_async_copy(
                abuf, out_hbm.at[pl.ds(base, SUB), pl.ds(c * DH, DH)],
                semi1).wait()

        pltpu.async_copy(acc.at[pl.ds(base, SUB)], abuf, semg0)
        pltpu.make_async_copy(acc.at[pl.ds(base, SUB)], abuf, semg0).wait()
        pltpu.make_async_copy(cnt_src.at[pl.ds(base, SUB)], cslc, semi0).wait()
        pltpu.make_async_copy(zl.at[pl.ds(base, SUB)], zbuf, semg1).wait()

        def grp_g(g, carry2):
            invv = 1.0 / jnp.maximum(cbuf[pl.ds(g * 16, 16)], 1.0)
            for r in range(16):
                iv = jnp.broadcast_to(invv[r], (16,))
                for v in range(DH // 16):
                    sl = pl.ds(v * 16, 16)
                    val = buf0[g * 16 + r, sl] * iv + buf1[g * 16 + r, sl]
                    if relu:
                        val = jnp.maximum(val, 0.0)
                    buf0[g * 16 + r, sl] = val
            return carry2
        lax.fori_loop(0, SUB // 16, grp_g, 0)
        pltpu.async_copy(abuf, out_hbm.at[pl.ds(base, SUB), pl.ds(c * DH, DH)],
                         semi1)
        return carry
    lax.fori_loop(0, nsub, out_j, 0)
    pltpu.make_async_copy(
        abuf, out_hbm.at[pl.ds(row0 + (nsub - 1) * SUB, SUB),
                         pl.ds(c * DH, DH)], semi1).wait()


def _make_spmm(do_counts):
    mesh = plsc.VectorSubcoreMesh(core_axis_name="c", subcore_axis_name="s")
    scratch = [
        pltpu.VMEM((NKP * CHUNK,), jnp.int32),
        pltpu.VMEM((2, CHUNK), jnp.int32),
        pltpu.VMEM((CHUNK, DH), jnp.float32),
        pltpu.VMEM((CHUNK, DH), jnp.float32),
    ]
    if do_counts:
        scratch.append(pltpu.VMEM((CHUNK,), jnp.float32))
    scratch.append(pltpu.VMEM((SUB + 16,), jnp.float32))
    scratch.append(pltpu.VMEM_SHARED((N, DH), jnp.float32))
    if do_counts:
        scratch.append(pltpu.VMEM_SHARED((N,), jnp.float32))
    scratch += [pltpu.SemaphoreType.DMA] * 4
    if do_counts:
        out_type = (jax.ShapeDtypeStruct((N, D), jnp.float32),
                    jax.ShapeDtypeStruct((N,), jnp.float32))
    else:
        out_type = jax.ShapeDtypeStruct((N, D), jnp.float32)
    return pl.kernel(
        lambda *refs: _spmm_body(do_counts, refs),
        out_type=out_type,
        mesh=mesh,
        scratch_types=scratch,
    )


def kernel(x, edge_index, W_l0, b_l0, W_r0, W_l1, b_l1, W_r1):
    src = edge_index[0]
    dst = edge_index[1].reshape(NCHUNKS, CHUNK)
    zrow = jnp.zeros((ROWS_PER_TILE, DH), jnp.float32)
    zcnt = jnp.zeros((N,), jnp.float32)
    ones = jnp.ones((CHUNK,), jnp.float32)

    yl, yr, zl, zr = _make_prep()(x, W_l0, W_r0, b_l0.reshape(1, D))
    h, cnt = _make_spmm(True)(yl, yr, zl, zr, src, dst, zrow, zcnt, ones)
    yl1, yr1, zl1, zr1 = _make_prep()(h, W_l1, W_r1, b_l1.reshape(1, D))
    out = _make_spmm(False)(yl1, yr1, zl1, zr1, src, dst, zrow, cnt)
    return out


# speedup vs baseline: 8.6464x; 1.0053x over previous
import functools

import jax
import jax.numpy as jnp
from jax import lax
from jax.experimental import pallas as pl
from jax.experimental.pallas import tpu as pltpu
from jax.experimental.pallas import tpu_sc as plsc

N = 10000
E = 160000
D = 256
DH = D // 2
ROWS_PER_TILE = 640
SUB = 80
CHUNK = 128
NCHUNKS = E // CHUNK
NKP = 80
NBLK = 2
BLK = N // NBLK


def _prep_compute(x, wl_ref, wr_ref, b_ref, yl_ref, yr_ref, zl_ref, zr_ref):
    y = jnp.dot(x, wl_ref[...], preferred_element_type=jnp.float32)
    z = jnp.dot(x, wr_ref[...], preferred_element_type=jnp.float32) + b_ref[...]
    yl_ref[...] = y[:, :DH]
    yr_ref[...] = y[:, DH:]
    zl_ref[...] = z[:, :DH]
    zr_ref[...] = z[:, DH:]


def _prep_body_full(x_ref, wl_ref, wr_ref, b_ref, *out_refs):
    _prep_compute(x_ref[...], wl_ref, wr_ref, b_ref, *out_refs)


def _make_prep():
    half = pl.BlockSpec((BLK, DH), lambda i: (i, 0))
    return pl.pallas_call(
        _prep_body_full,
        grid=(NBLK,),
        in_specs=[
            pl.BlockSpec((BLK, D), lambda i: (i, 0)),
            pl.BlockSpec((D, D), lambda i: (0, 0)),
            pl.BlockSpec((D, D), lambda i: (0, 0)),
            pl.BlockSpec((1, D), lambda i: (0, 0)),
        ],
        out_specs=[half, half, half, half],
        out_shape=[jax.ShapeDtypeStruct((N, DH), jnp.float32)] * 4,
    )


def _spmm_body(do_counts, refs):
    relu = do_counts
    if do_counts:
        (yl, yr, zl, zr, src_hbm, dst_hbm, zrow, zcnt, ones_hbm,
         out_hbm, cnt_out, src_v, dring, buf0, buf1, ones_v, cbuf, acc, cnt,
         semg0, semg1, semi0, semi1) = refs
    else:
        (yl, yr, zl, zr, src_hbm, dst_hbm, zrow, cnt_hbm,
         out_hbm, src_v, dring, buf0, buf1, cbuf, acc,
         semg0, semg1, semi0, semi1) = refs
    c = lax.axis_index("c")
    s = lax.axis_index("s")
    row0 = s * ROWS_PER_TILE
    nrows = jnp.minimum(ROWS_PER_TILE, N - row0)
    nsub = nrows // SUB

    @pl.when(s < 15)
    def _():
        pltpu.sync_copy(zrow, acc.at[pl.ds(row0, ROWS_PER_TILE)])

    @pl.when(s == 15)
    def _():
        pltpu.sync_copy(zrow.at[pl.ds(0, N - 15 * ROWS_PER_TILE)],
                        acc.at[pl.ds(15 * ROWS_PER_TILE,
                                     N - 15 * ROWS_PER_TILE)])

    if do_counts:
        @pl.when(s == 0)
        def _():
            pltpu.sync_copy(zcnt, cnt)

        pltpu.sync_copy(ones_hbm, ones_v)
    plsc.subcore_barrier()

    start = jnp.minimum(s * NKP, NCHUNKS - NKP)
    klo = jnp.maximum(s * NKP - (NCHUNKS - NKP), 0)
    pltpu.sync_copy(src_hbm.at[pl.ds(start * CHUNK, NKP * CHUNK)], src_v)

    bufs = (buf0, buf1)
    semg = (semg0, semg1)
    semi = (semi0, semi1)

    def fire(k, b):
        pltpu.async_copy(dst_hbm.at[start + k], dring.at[b], semi[b])

        @pl.when(c == 0)
        def _():
            pltpu.async_copy(yl.at[src_v.at[pl.ds(k * CHUNK, CHUNK)]],
                             bufs[b], semg[b])

        @pl.when(c == 1)
        def _():
            pltpu.async_copy(yr.at[src_v.at[pl.ds(k * CHUNK, CHUNK)]],
                             bufs[b], semg[b])

    for b in range(2):
        @pl.when(klo + b < NKP)
        def _(b=b):
            fire(klo + b, b)

    def pair_k(k2, carry):
        for b in range(2):
            k = klo + k2 * 2 + b

            @pl.when(k < NKP)
            def _(k=k, b=b):
                pltpu.make_async_copy(dst_hbm.at[0], dring.at[b], semi[b]).wait()
                pltpu.make_async_copy(yl.at[src_v.at[pl.ds(0, CHUNK)]],
                                      bufs[b], semg[b]).wait()
                pltpu.sync_copy(bufs[b], acc.at[dring.at[b]], add=True)
                if do_counts:
                    pltpu.sync_copy(ones_v, cnt.at[dring.at[b]], add=True)

                @pl.when(k + 2 < NKP)
                def _():
                    fire(k + 2, b)
        return carry
    lax.fori_loop(0, NKP // 2, pair_k, 0)
    plsc.subcore_barrier()

    if do_counts:
        @pl.when(jnp.logical_and(s == 0, c == 0))
        def _():
            pltpu.sync_copy(cnt, cnt_out)

    abuf = buf0.at[pl.ds(0, SUB)]
    zbuf = buf1.at[pl.ds(0, SUB)]
    cslc = cbuf.at[pl.ds(0, SUB)]

    def out_j(j, carry):
        base = row0 + j * SUB
        @pl.when(c == 0)
        def _():
            pltpu.async_copy(zl.at[pl.ds(base, SUB)], zbuf, semg1)

        @pl.when(c == 1)
        def _():
            pltpu.async_copy(zr.at[pl.ds(base, SUB)], zbuf, semg1)

        cnt_src = cnt if do_counts else cnt_hbm
        pltpu.async_copy(cnt_src.at[pl.ds(base, SUB)], cslc, semi0)

        @pl.when(j > 0)
        def _():
            pltpu.make_async_copy(
                abuf, out_hbm.at[pl.ds(base, SUB), pl.ds(c * DH, DH)],
                semi1).wait()

        pltpu.async_copy(acc.at[pl.ds(base, SUB)], abuf, semg0)
        pltpu.make_async_copy(acc.at[pl.ds(base, SUB)], abuf, semg0).wait()
        pltpu.make_async_copy(cnt_src.at[pl.ds(base, SUB)], cslc, semi0).wait()
        pltpu.make_async_copy(zl.at[pl.ds(base, SUB)], zbuf, semg1).wait()

        def grp_g(g, carry2):
            invv = 1.0 / jnp.maximum(cbuf[pl.ds(g * 16, 16)], 1.0)
            for r in range(16):
                iv = jnp.broadcast_to(invv[r], (16,))
                for v in range(DH // 16):
                    sl = pl.ds(v * 16, 16)
                    val = buf0[g * 16 + r, sl] * iv + buf1[g * 16 + r, sl]
                    if relu:
                        val = jnp.maximum(val, 0.0)
                    buf0[g * 16 + r, sl] = val
            return carry2
        lax.fori_loop(0, SUB // 16, grp_g, 0)
        pltpu.async_copy(abuf, out_hbm.at[pl.ds(base, SUB), pl.ds(c * DH, DH)],
                         semi1)
        return carry
    lax.fori_loop(0, nsub, out_j, 0)
    pltpu.make_async_copy(
        abuf, out_hbm.at[pl.ds(row0 + (nsub - 1) * SUB, SUB),
                         pl.ds(c * DH, DH)], semi1).wait()


def _make_spmm(do_counts):
    mesh = plsc.VectorSubcoreMesh(core_axis_name="c", subcore_axis_name="s")
    scratch = [
        pltpu.VMEM((NKP * CHUNK,), jnp.int32),
        pltpu.VMEM((2, CHUNK), jnp.int32),
        pltpu.VMEM((CHUNK, DH), jnp.float32),
        pltpu.VMEM((CHUNK, DH), jnp.float32),
    ]
    if do_counts:
        scratch.append(pltpu.VMEM((CHUNK,), jnp.float32))
    scratch.append(pltpu.VMEM((SUB + 16,), jnp.float32))
    scratch.append(pltpu.VMEM_SHARED((N, DH), jnp.float32))
    if do_counts:
        scratch.append(pltpu.VMEM_SHARED((N,), jnp.float32))
    scratch += [pltpu.SemaphoreType.DMA] * 4
    if do_counts:
        out_type = (jax.ShapeDtypeStruct((N, D), jnp.float32),
                    jax.ShapeDtypeStruct((N,), jnp.float32))
    else:
        out_type = jax.ShapeDtypeStruct((N, D), jnp.float32)
    return pl.kernel(
        lambda *refs: _spmm_body(do_counts, refs),
        out_type=out_type,
        mesh=mesh,
        scratch_types=scratch,
    )


def kernel(x, edge_index, W_l0, b_l0, W_r0, W_l1, b_l1, W_r1):
    src = edge_index[0]
    dst = edge_index[1].reshape(NCHUNKS, CHUNK)
    zrow = jnp.zeros((ROWS_PER_TILE, DH), jnp.float32)
    zcnt = jnp.zeros((N,), jnp.float32)
    ones = jnp.ones((CHUNK,), jnp.float32)

    yl, yr, zl, zr = _make_prep()(x, W_l0, W_r0, b_l0.reshape(1, D))
    h, cnt = _make_spmm(True)(yl, yr, zl, zr, src, dst, zrow, zcnt, ones)
    yl1, yr1, zl1, zr1 = _make_prep()(h, W_l1, W_r1, b_l1.reshape(1, D))
    out = _make_spmm(False)(yl1, yr1, zl1, zr1, src, dst, zrow, cnt)
    return out
